# baseline (speedup 1.0000x reference)
import jax
import jax.numpy as jnp
from jax.experimental import pallas as pl
from jax.experimental.pallas import tpu as pltpu

_L_FIXED = 200
_THREE_L = 3 * _L_FIXED
_TILE_B = 64


def _body(off_ref, table_ref, out_ref, scratch, sem):
    @pl.when(pl.program_id(0) == 0)
    def _fill():
        off = pl.multiple_of(off_ref[0], 8)
        gather = pltpu.make_async_copy(
            table_ref.at[pl.ds(off, _THREE_L), :], scratch.at[0], sem
        )
        gather.start()
        gather.wait()
        n = 1
        while n < _TILE_B:
            m = min(n, _TILE_B - n)
            dbl = pltpu.make_async_copy(
                scratch.at[pl.ds(0, m)], scratch.at[pl.ds(n, m)], sem
            )
            dbl.start()
            dbl.wait()
            n += m

    cp = pltpu.make_async_copy(scratch, out_ref, sem)
    cp.start()
    cp.wait()


def kernel(timesteps, L, table):
    batch = timesteps.shape[0]
    d = table.shape[1]
    offset = jnp.asarray(L - _L_FIXED, jnp.int32).reshape(1)
    out = pl.pallas_call(
        _body,
        grid_spec=pltpu.PrefetchScalarGridSpec(
            num_scalar_prefetch=1,
            grid=(batch // _TILE_B,),
            in_specs=[pl.BlockSpec(memory_space=pl.ANY)],
            out_specs=pl.BlockSpec(
                (_TILE_B, _THREE_L, d), lambda i, off: (i, 0, 0)
            ),
            scratch_shapes=[
                pltpu.VMEM((_TILE_B, _THREE_L, d), table.dtype),
                pltpu.SemaphoreType.DMA,
            ],
        ),
        out_shape=jax.ShapeDtypeStruct((batch, _THREE_L, d), table.dtype),
    )(offset, table)
    return out


# speedup vs baseline: 10.0708x; 10.0708x over previous
import jax
import jax.numpy as jnp
from jax.experimental import pallas as pl
from jax.experimental.pallas import tpu as pltpu

_L_FIXED = 200
_THREE_L = 3 * _L_FIXED
_TILE_T = 75


def _body(off_ref, table_ref, out_ref, emb_ref, sem):
    i = pl.program_id(0)

    @pl.when(i == 0)
    def _gather():
        off = pl.multiple_of(off_ref[0], 8)
        cp = pltpu.make_async_copy(
            table_ref.at[pl.ds(off, _THREE_L), :], emb_ref, sem
        )
        cp.start()
        cp.wait()

    blk = emb_ref[pl.ds(i * _TILE_T, _TILE_T), :]
    out_ref[...] = jnp.broadcast_to(blk[:, :, None], out_ref.shape)


def kernel(timesteps, L, table):
    batch = timesteps.shape[0]
    d = table.shape[1]
    offset = jnp.asarray(L - _L_FIXED, jnp.int32).reshape(1)
    tmp = pl.pallas_call(
        _body,
        grid_spec=pltpu.PrefetchScalarGridSpec(
            num_scalar_prefetch=1,
            grid=(_THREE_L // _TILE_T,),
            in_specs=[pl.BlockSpec(memory_space=pl.ANY)],
            out_specs=pl.BlockSpec(
                (_TILE_T, d, batch), lambda i, off: (i, 0, 0)
            ),
            scratch_shapes=[
                pltpu.VMEM((_THREE_L, d), table.dtype),
                pltpu.SemaphoreType.DMA,
            ],
        ),
        out_shape=jax.ShapeDtypeStruct((_THREE_L, d, batch), table.dtype),
    )(offset, table)
    return tmp.transpose(2, 0, 1)


# speedup vs baseline: 10.3479x; 1.0275x over previous
import jax
import jax.numpy as jnp
from jax.experimental import pallas as pl
from jax.experimental.pallas import tpu as pltpu

_L_FIXED = 200
_THREE_L = 3 * _L_FIXED
_TILE_T = 40


def _body(off_ref, table_ref, out_ref):
    i = pl.program_id(0)
    start = pl.multiple_of(off_ref[0] + i * _TILE_T, 8)
    blk = table_ref[pl.ds(start, _TILE_T), :]
    out_ref[...] = jnp.broadcast_to(blk[:, :, None], out_ref.shape)


def kernel(timesteps, L, table):
    batch = timesteps.shape[0]
    rows, d = table.shape
    offset = jnp.asarray(L - _L_FIXED, jnp.int32).reshape(1)
    tmp = pl.pallas_call(
        _body,
        grid_spec=pltpu.PrefetchScalarGridSpec(
            num_scalar_prefetch=1,
            grid=(_THREE_L // _TILE_T,),
            in_specs=[pl.BlockSpec((rows, d), lambda i, off: (0, 0))],
            out_specs=pl.BlockSpec(
                (_TILE_T, d, batch), lambda i, off: (i, 0, 0)
            ),
        ),
        out_shape=jax.ShapeDtypeStruct((_THREE_L, d, batch), table.dtype),
    )(offset, table)
    return tmp.transpose(2, 0, 1)
